# Initial kernel scaffold; baseline (speedup 1.0000x reference)
import functools

import jax
import jax.numpy as jnp
from jax import lax
from jax.experimental import pallas as pl
from jax.experimental.pallas import tpu as pltpu
from jax.experimental.pallas import tpu_sc as plsc

N = 10000
E = 320000
H = 128
G = 64
C = 10
EPS = 1e-5

NC = 2
NS = 16
NW = NC * NS
K = 128
CPT = 79
E_PAD = NW * CPT * K
NACC = 10240
RPT = NACC // NS
ZB = 64
DUMP = NACC - 1

_mesh = plsc.VectorSubcoreMesh(
    core_axis_name="c", subcore_axis_name="s", num_cores=NC, num_subcores=NS
)


@functools.partial(
    pl.kernel,
    out_type=(
        jax.ShapeDtypeStruct((NACC, 16), jnp.float32),
        jax.ShapeDtypeStruct((NACC, 16), jnp.float32),
    ),
    mesh=_mesh,
    scratch_types=[
        pltpu.VMEM((CPT, K), jnp.int32),
        pltpu.VMEM((K, 16), jnp.float32),
        pltpu.VMEM((RPT, 16), jnp.float32),
        pltpu.VMEM_SHARED((NACC, 16), jnp.float32),
    ],
)
def _sc_degree(col_hbm, out0, out1, col_v, ones_v, z_v, acc):
    cid = lax.axis_index("c")
    sid = lax.axis_index("s")
    wid = cid * NS + sid

    @pl.loop(0, K)
    def _(j):
        ones_v[j, :] = jnp.ones((16,), jnp.float32)

    @pl.loop(0, RPT)
    def _(j):
        z_v[j, :] = jnp.zeros((16,), jnp.float32)

    pltpu.sync_copy(z_v, acc.at[pl.ds(sid * RPT, RPT)])
    plsc.subcore_barrier()

    pltpu.sync_copy(col_hbm.at[wid], col_v)

    @pl.loop(0, CPT)
    def _(j):
        pltpu.sync_copy(ones_v, acc.at[col_v.at[j]], add=True)

    plsc.subcore_barrier()

    @pl.when(cid == 0)
    def _():
        pltpu.sync_copy(acc.at[pl.ds(sid * RPT, RPT)], out0.at[pl.ds(sid * RPT, RPT)])

    @pl.when(cid == 1)
    def _():
        pltpu.sync_copy(acc.at[pl.ds(sid * RPT, RPT)], out1.at[pl.ds(sid * RPT, RPT)])


@functools.partial(
    pl.kernel,
    out_type=(
        jax.ShapeDtypeStruct((NACC, H), jnp.float32),
        jax.ShapeDtypeStruct((NACC, H), jnp.float32),
    ),
    mesh=_mesh,
    scratch_types=[
        pltpu.VMEM((CPT, K), jnp.int32),
        pltpu.VMEM((CPT, K), jnp.int32),
        pltpu.VMEM((K, H), jnp.float32),
        pltpu.VMEM((ZB, H), jnp.float32),
        pltpu.VMEM_SHARED((NACC, H), jnp.float32),
        pltpu.SemaphoreType.DMA,
    ],
)
def _sc_aggregate(xs_hbm, row_hbm, col_hbm, out0, out1, row_v, col_v, buf, z_v,
                  acc, sem):
    cid = lax.axis_index("c")
    sid = lax.axis_index("s")
    wid = cid * NS + sid

    @pl.loop(0, ZB)
    def _(j):
        @pl.loop(0, H, step=16)
        def _(l):
            z_v[j, pl.ds(l, 16)] = jnp.zeros((16,), jnp.float32)

    @pl.loop(0, RPT, step=ZB)
    def _(r):
        pltpu.sync_copy(z_v, acc.at[pl.ds(sid * RPT + r, ZB)])

    plsc.subcore_barrier()

    pltpu.sync_copy(row_hbm.at[wid], row_v)
    pltpu.sync_copy(col_hbm.at[wid], col_v)

    @pl.loop(0, CPT)
    def _(j):
        pltpu.async_copy(xs_hbm.at[row_v.at[j]], buf, sem).wait()
        pltpu.sync_copy(buf, acc.at[col_v.at[j]], add=True)

    plsc.subcore_barrier()

    @pl.when(cid == 0)
    def _():
        pltpu.sync_copy(acc.at[pl.ds(sid * RPT, RPT)], out0.at[pl.ds(sid * RPT, RPT)])

    @pl.when(cid == 1)
    def _():
        pltpu.sync_copy(acc.at[pl.ds(sid * RPT, RPT)], out1.at[pl.ds(sid * RPT, RPT)])


_BLK = 1000
_NBLK = N // _BLK


def _tc_dinv(d0, d1):
    def body(d0_ref, d1_ref, o_ref):
        deg = d0_ref[:, 0:1] + d1_ref[:, 0:1] + 1.0
        o_ref[...] = lax.rsqrt(deg)

    return pl.pallas_call(
        body,
        out_shape=jax.ShapeDtypeStruct((NACC, 1), jnp.float32),
    )(d0, d1)


def _tc_matmul_scale(h, W, dinv):
    def body(h_ref, w_ref, d_ref, o_ref):
        o_ref[...] = d_ref[...] * jnp.dot(
            h_ref[...], w_ref[...], preferred_element_type=jnp.float32
        )

    return pl.pallas_call(
        body,
        grid=(_NBLK,),
        in_specs=[
            pl.BlockSpec((_BLK, H), lambda i: (i, 0)),
            pl.BlockSpec((H, H), lambda i: (0, 0)),
            pl.BlockSpec((_BLK, 1), lambda i: (i, 0)),
        ],
        out_specs=pl.BlockSpec((_BLK, H), lambda i: (i, 0)),
        out_shape=jax.ShapeDtypeStruct((N, H), jnp.float32),
    )(h, W, dinv)


def _tc_combine(p0, p1, xs, dinv, b, g, be):
    def body(p0_ref, p1_ref, xs_ref, d_ref, b_ref, g_ref, be_ref, o_ref):
        gg = g_ref[...] * lax.rsqrt(jnp.float32(1.0 + EPS))
        agg = d_ref[...] * (p0_ref[...] + p1_ref[...] + xs_ref[...]) + b_ref[...]
        o_ref[...] = jnp.maximum(gg * agg + be_ref[...], 0.0)

    return pl.pallas_call(
        body,
        grid=(_NBLK,),
        in_specs=[
            pl.BlockSpec((_BLK, H), lambda i: (i, 0)),
            pl.BlockSpec((_BLK, H), lambda i: (i, 0)),
            pl.BlockSpec((_BLK, H), lambda i: (i, 0)),
            pl.BlockSpec((_BLK, 1), lambda i: (i, 0)),
            pl.BlockSpec((1, H), lambda i: (0, 0)),
            pl.BlockSpec((1, H), lambda i: (0, 0)),
            pl.BlockSpec((1, H), lambda i: (0, 0)),
        ],
        out_specs=pl.BlockSpec((_BLK, H), lambda i: (i, 0)),
        out_shape=jax.ShapeDtypeStruct((N, H), jnp.float32),
    )(p0, p1, xs, dinv, b.reshape(1, H), g.reshape(1, H), be.reshape(1, H))


def _tc_pool_head(h, batch2d, lw1, lb1, g4, be4, lw2, lb2):
    def body(h_ref, b_ref, lw1_ref, lb1_ref, g4_ref, be4_ref, lw2_ref, lb2_ref,
             o_ref, sums, cnt):
        i = pl.program_id(0)

        @pl.when(i == 0)
        def _():
            sums[...] = jnp.zeros_like(sums)
            cnt[...] = jnp.zeros_like(cnt)

        seg = b_ref[...]
        onehot = (seg == lax.broadcasted_iota(jnp.int32, (_BLK, G), 1)).astype(
            jnp.float32
        )
        sums[...] += lax.dot_general(
            onehot, h_ref[...], (((0,), (0,)), ((), ())),
            preferred_element_type=jnp.float32,
        )
        cnt[...] += jnp.sum(onehot, axis=0, keepdims=True)

        @pl.when(i == _NBLK - 1)
        def _():
            p = sums[...] / jnp.maximum(cnt[...], 1.0).T
            gg = g4_ref[...] * lax.rsqrt(jnp.float32(1.0 + EPS))
            q = jnp.dot(p, lw1_ref[...], preferred_element_type=jnp.float32)
            q = jnp.maximum(gg * (q + lb1_ref[...]) + be4_ref[...], 0.0)
            o_ref[...] = (
                jnp.dot(q, lw2_ref[...], preferred_element_type=jnp.float32)
                + lb2_ref[...]
            )

    return pl.pallas_call(
        body,
        grid=(_NBLK,),
        in_specs=[
            pl.BlockSpec((_BLK, H), lambda i: (i, 0)),
            pl.BlockSpec((_BLK, 1), lambda i: (i, 0)),
            pl.BlockSpec((H, H), lambda i: (0, 0)),
            pl.BlockSpec((1, H), lambda i: (0, 0)),
            pl.BlockSpec((1, H), lambda i: (0, 0)),
            pl.BlockSpec((1, H), lambda i: (0, 0)),
            pl.BlockSpec((H, C), lambda i: (0, 0)),
            pl.BlockSpec((1, C), lambda i: (0, 0)),
        ],
        out_specs=pl.BlockSpec((G, C), lambda i: (0, 0)),
        out_shape=jax.ShapeDtypeStruct((G, C), jnp.float32),
        scratch_shapes=[
            pltpu.VMEM((G, H), jnp.float32),
            pltpu.VMEM((1, G), jnp.float32),
        ],
    )(h, batch2d, lw1, lb1.reshape(1, H), g4.reshape(1, H), be4.reshape(1, H),
      lw2, lb2.reshape(1, C))


def kernel(x, edge_index, batch, W1, b1, g1, be1, W2, b2, g2, be2,
           W3, b3, g3, be3, lw1, lb1, g4, be4, lw2, lb2):
    row = edge_index[0]
    col = edge_index[1]
    pad = E_PAD - E
    row3 = jnp.concatenate([row, jnp.zeros((pad,), jnp.int32)]).reshape(NW, CPT, K)
    col3 = jnp.concatenate([col, jnp.full((pad,), DUMP, jnp.int32)]).reshape(NW, CPT, K)

    d0, d1 = _sc_degree(col3)
    dinv = _tc_dinv(d0, d1)

    h = x
    for W, b, g, be in ((W1, b1, g1, be1), (W2, b2, g2, be2), (W3, b3, g3, be3)):
        xs = _tc_matmul_scale(h, W, dinv)
        p0, p1 = _sc_aggregate(xs, row3, col3)
        h = _tc_combine(p0, p1, xs, dinv, b, g, be)

    return _tc_pool_head(h, batch.reshape(N, 1), lw1, lb1, g4, be4, lw2, lb2)


# speedup vs baseline: 9.8346x; 9.8346x over previous
import dataclasses
import functools

import jax
import jax.numpy as jnp
from jax import lax
from jax.experimental import pallas as pl
from jax.experimental.pallas import tpu as pltpu
from jax.experimental.pallas import tpu_sc as plsc

N = 10000
E = 320000
H = 128
G = 64
C = 10
EPS = 1e-5

NC = 2
NS = 16
NW = NC * NS
K = 128
CPT = 79
E_PAD = NW * CPT * K
NP = 10240
RPT = NP // NS
ZB = 64
DUMP = NP - 1

_mesh = plsc.VectorSubcoreMesh(
    core_axis_name="c", subcore_axis_name="s", num_cores=NC, num_subcores=NS
)
_sc_no_layout = dataclasses.replace(
    pltpu.CompilerParams(), needs_layout_passes=False
)


@functools.partial(
    pl.kernel,
    out_type=jax.ShapeDtypeStruct((NW, NP), jnp.float32),
    mesh=_mesh,
    scratch_types=[
        pltpu.VMEM((K,), jnp.int32),
        pltpu.VMEM((NP,), jnp.float32),
    ],
    compiler_params=_sc_no_layout,
)
def _sc_degree(col_hbm, out, colbuf, hist_v):
    cid = lax.axis_index("c")
    sid = lax.axis_index("s")
    wid = cid * NS + sid

    @pl.loop(0, NP, step=16)
    def _(r):
        hist_v[pl.ds(r, 16)] = jnp.zeros((16,), jnp.float32)

    ones16 = jnp.ones((16,), jnp.float32)

    @pl.loop(0, CPT)
    def _(j):
        pltpu.sync_copy(col_hbm.at[wid, j], colbuf)

        @pl.loop(0, K, step=16)
        def _(q):
            idx = colbuf[pl.ds(q, 16)]
            plsc.addupdate_scatter(hist_v, [idx], ones16)

    pltpu.sync_copy(hist_v, out.at[wid])


@functools.partial(
    pl.kernel,
    out_type=jax.ShapeDtypeStruct((2 * NP, H), jnp.float32),
    mesh=_mesh,
    scratch_types=[
        pltpu.VMEM((K,), jnp.int32),
        pltpu.VMEM((K,), jnp.int32),
        pltpu.VMEM((K, H), jnp.float32),
        pltpu.VMEM((ZB, H), jnp.float32),
        pltpu.VMEM_SHARED((NP, H), jnp.float32),
        pltpu.SemaphoreType.DMA,
    ],
)
def _sc_aggregate(xs_hbm, row_hbm, col_hbm, zero_hbm, out, rowbuf, colbuf,
                  buf, z_v, acc, sem):
    cid = lax.axis_index("c")
    sid = lax.axis_index("s")
    wid = cid * NS + sid

    pltpu.sync_copy(zero_hbm, z_v)

    @pl.loop(0, RPT, step=ZB)
    def _(r):
        pltpu.sync_copy(z_v, acc.at[pl.ds(sid * RPT + r, ZB)])

    plsc.subcore_barrier()

    @pl.loop(0, CPT)
    def _(j):
        pltpu.sync_copy(row_hbm.at[wid, j], rowbuf)
        pltpu.sync_copy(col_hbm.at[wid, j], colbuf)
        pltpu.async_copy(xs_hbm.at[rowbuf], buf, sem).wait()
        pltpu.sync_copy(buf, acc.at[colbuf], add=True)

    plsc.subcore_barrier()

    @pl.loop(0, RPT, step=ZB)
    def _(r):
        pltpu.sync_copy(acc.at[pl.ds(sid * RPT + r, ZB)], z_v)
        pltpu.sync_copy(z_v, out.at[pl.ds(cid * NP + sid * RPT + r, ZB)])


_BLK = 1024
_NBLK = NP // _BLK


def _tc_dinv(degs):
    def body(d_ref, o_ref):
        s = jnp.sum(d_ref[...], axis=0, keepdims=True) + 1.0
        o_ref[...] = jnp.transpose(lax.rsqrt(s))

    return pl.pallas_call(
        body,
        out_shape=jax.ShapeDtypeStruct((NP, 1), jnp.float32),
    )(degs)


def _tc_matmul_scale(h, W, dinv):
    def body(h_ref, w_ref, d_ref, o_ref):
        o_ref[...] = d_ref[...] * jnp.dot(
            h_ref[...], w_ref[...], preferred_element_type=jnp.float32
        )

    return pl.pallas_call(
        body,
        grid=(_NBLK,),
        in_specs=[
            pl.BlockSpec((_BLK, H), lambda i: (i, 0)),
            pl.BlockSpec((H, H), lambda i: (0, 0)),
            pl.BlockSpec((_BLK, 1), lambda i: (i, 0)),
        ],
        out_specs=pl.BlockSpec((_BLK, H), lambda i: (i, 0)),
        out_shape=jax.ShapeDtypeStruct((NP, H), jnp.float32),
    )(h, W, dinv)


def _tc_combine(p, xs, dinv, b, g, be):
    def body(p0_ref, p1_ref, xs_ref, d_ref, b_ref, g_ref, be_ref, o_ref):
        gg = g_ref[...] * lax.rsqrt(jnp.float32(1.0 + EPS))
        agg = d_ref[...] * (p0_ref[...] + p1_ref[...] + xs_ref[...]) + b_ref[...]
        o_ref[...] = jnp.maximum(gg * agg + be_ref[...], 0.0)

    return pl.pallas_call(
        body,
        grid=(_NBLK,),
        in_specs=[
            pl.BlockSpec((_BLK, H), lambda i: (i, 0)),
            pl.BlockSpec((_BLK, H), lambda i: (_NBLK + i, 0)),
            pl.BlockSpec((_BLK, H), lambda i: (i, 0)),
            pl.BlockSpec((_BLK, 1), lambda i: (i, 0)),
            pl.BlockSpec((1, H), lambda i: (0, 0)),
            pl.BlockSpec((1, H), lambda i: (0, 0)),
            pl.BlockSpec((1, H), lambda i: (0, 0)),
        ],
        out_specs=pl.BlockSpec((_BLK, H), lambda i: (i, 0)),
        out_shape=jax.ShapeDtypeStruct((NP, H), jnp.float32),
    )(p, p, xs, dinv, b.reshape(1, H), g.reshape(1, H), be.reshape(1, H))


def _tc_pool_head(h, batch2d, lw1, lb1, g4, be4, lw2, lb2):
    def body(h_ref, b_ref, lw1_ref, lb1_ref, g4_ref, be4_ref, lw2_ref, lb2_ref,
             o_ref, sums, cnt):
        i = pl.program_id(0)

        @pl.when(i == 0)
        def _():
            sums[...] = jnp.zeros_like(sums)
            cnt[...] = jnp.zeros_like(cnt)

        seg = b_ref[...]
        onehot = (seg == lax.broadcasted_iota(jnp.int32, (_BLK, G), 1)).astype(
            jnp.float32
        )
        sums[...] += lax.dot_general(
            onehot, h_ref[...], (((0,), (0,)), ((), ())),
            preferred_element_type=jnp.float32,
        )
        cnt[...] += jnp.sum(onehot, axis=0, keepdims=True)

        @pl.when(i == _NBLK - 1)
        def _():
            p = sums[...] / jnp.maximum(cnt[...], 1.0).T
            gg = g4_ref[...] * lax.rsqrt(jnp.float32(1.0 + EPS))
            q = jnp.dot(p, lw1_ref[...], preferred_element_type=jnp.float32)
            q = jnp.maximum(gg * (q + lb1_ref[...]) + be4_ref[...], 0.0)
            o_ref[...] = (
                jnp.dot(q, lw2_ref[...], preferred_element_type=jnp.float32)
                + lb2_ref[...]
            )

    return pl.pallas_call(
        body,
        grid=(_NBLK,),
        in_specs=[
            pl.BlockSpec((_BLK, H), lambda i: (i, 0)),
            pl.BlockSpec((_BLK, 1), lambda i: (i, 0)),
            pl.BlockSpec((H, H), lambda i: (0, 0)),
            pl.BlockSpec((1, H), lambda i: (0, 0)),
            pl.BlockSpec((1, H), lambda i: (0, 0)),
            pl.BlockSpec((1, H), lambda i: (0, 0)),
            pl.BlockSpec((H, C), lambda i: (0, 0)),
            pl.BlockSpec((1, C), lambda i: (0, 0)),
        ],
        out_specs=pl.BlockSpec((G, C), lambda i: (0, 0)),
        out_shape=jax.ShapeDtypeStruct((G, C), jnp.float32),
        scratch_shapes=[
            pltpu.VMEM((G, H), jnp.float32),
            pltpu.VMEM((1, G), jnp.float32),
        ],
    )(h, batch2d, lw1, lb1.reshape(1, H), g4.reshape(1, H), be4.reshape(1, H),
      lw2, lb2.reshape(1, C))


def kernel(x, edge_index, batch, W1, b1, g1, be1, W2, b2, g2, be2,
           W3, b3, g3, be3, lw1, lb1, g4, be4, lw2, lb2):
    row = edge_index[0]
    col = edge_index[1]
    pad = E_PAD - E
    row3 = jnp.concatenate([row, jnp.zeros((pad,), jnp.int32)]).reshape(NW, CPT, K)
    col3 = jnp.concatenate([col, jnp.full((pad,), DUMP, jnp.int32)]).reshape(NW, CPT, K)
    x_p = jnp.concatenate([x, jnp.zeros((NP - N, x.shape[1]), x.dtype)])
    batch2d = jnp.concatenate([batch, jnp.full((NP - N,), G, batch.dtype)])
    batch2d = batch2d.reshape(NP, 1)
    zrows = jnp.zeros((ZB, H), jnp.float32)

    degs = _sc_degree(col3)
    dinv = _tc_dinv(degs)

    h = x_p
    for W, b, g, be in ((W1, b1, g1, be1), (W2, b2, g2, be2), (W3, b3, g3, be3)):
        xs = _tc_matmul_scale(h, W, dinv)
        p = _sc_aggregate(xs, row3, col3, zrows)
        h = _tc_combine(p, xs, dinv, b, g, be)

    return _tc_pool_head(h, batch2d, lw1, lb1, g4, be4, lw2, lb2)
